# baseline (device time: 24218 ns/iter reference)
import jax
import jax.numpy as jnp
from jax import lax
from jax.experimental import pallas as pl
from jax.experimental.pallas import tpu as pltpu


def kernel(Q, K, V):
    B, Sq, H, D = Q.shape
    _, Skv, _, _ = K.shape
    G = 4
    HG = H // G
    scale = D ** -0.5

    def body(q_hbm, k_hbm, v_hbm, out_ref,
             q_vmem, k_vmem, v_vmem, load_sems,
             pnum, pstats, nsend_sems, ssend_sems, nrecv_sems, srecv_sems):
        x = lax.axis_index("x")
        y = lax.axis_index("y")
        z = lax.axis_index("z")
        g = 2 * x + z

        barrier = pltpu.get_barrier_semaphore()
        peers = ((1 - x, y, z), (x, 1 - y, z), (x, y, 1 - z),
                 (1 - x, 1 - y, z), (1 - x, y, 1 - z), (x, 1 - y, 1 - z),
                 (1 - x, 1 - y, 1 - z))
        for nbr in peers:
            pl.semaphore_signal(barrier, inc=1, device_id=nbr,
                                device_id_type=pl.DeviceIdType.MESH)

        loads = []
        for h in range(HG):
            loads.append((
                pltpu.make_async_copy(
                    k_hbm.at[:, :, g * HG + h, :], k_vmem.at[h],
                    load_sems.at[h]),
                pltpu.make_async_copy(
                    v_hbm.at[:, :, g * HG + h, :], v_vmem.at[h],
                    load_sems.at[HG + h]),
                pltpu.make_async_copy(
                    q_hbm.at[:, :, g * HG + h, :], q_vmem.at[h],
                    load_sems.at[2 * HG + h]),
            ))
        for ld in loads:
            for c in ld:
                c.start()

        def compute(h):
            qb = q_vmem[h].astype(jnp.bfloat16)
            kb = k_vmem[h].astype(jnp.bfloat16)
            vb = v_vmem[h].astype(jnp.bfloat16)
            s = lax.dot_general(
                qb, kb, (((2,), (2,)), ((0,), (0,))),
                preferred_element_type=jnp.float32) * scale
            m = jnp.max(s, axis=-1)
            p = jnp.exp(s - m[..., None])
            l = jnp.sum(p, axis=-1)
            num = lax.dot_general(
                p.astype(jnp.bfloat16), vb, (((2,), (1,)), ((0,), (0,))),
                preferred_element_type=jnp.float32)
            pnum[y, g, h] = num.astype(jnp.bfloat16)
            pstats[y, g, h, 0] = m
            pstats[y, g, h, 1] = l

        HC = 2

        def start_sends(r):
            h0 = r * HC
            rdmas = []
            for j, dst in enumerate(peers):
                rn = pltpu.make_async_remote_copy(
                    pnum.at[y, g, pl.ds(h0, HC)],
                    pnum.at[y, g, pl.ds(h0, HC)],
                    nsend_sems.at[r, j], nrecv_sems.at[y, g, r],
                    device_id=dst, device_id_type=pl.DeviceIdType.MESH)
                rs = pltpu.make_async_remote_copy(
                    pstats.at[y, g, pl.ds(h0, HC)],
                    pstats.at[y, g, pl.ds(h0, HC)],
                    ssend_sems.at[r, j], srecv_sems.at[y, g, r],
                    device_id=dst, device_id_type=pl.DeviceIdType.MESH)
                rn.start()
                rs.start()
                rdmas.append((rn, rs))
            return rdmas

        def drain(r, sends):
            h0 = r * HC
            for yy in range(2):
                for gg in range(G):
                    is_self = jnp.logical_and(yy == y, gg == g)
                    rn = pltpu.make_async_remote_copy(
                        pnum.at[yy, gg, pl.ds(h0, HC)],
                        pnum.at[yy, gg, pl.ds(h0, HC)],
                        nsend_sems.at[r, 0], nrecv_sems.at[yy, gg, r],
                        device_id=(x, y, z),
                        device_id_type=pl.DeviceIdType.MESH)
                    rs = pltpu.make_async_remote_copy(
                        pstats.at[yy, gg, pl.ds(h0, HC)],
                        pstats.at[yy, gg, pl.ds(h0, HC)],
                        ssend_sems.at[r, 0], srecv_sems.at[yy, gg, r],
                        device_id=(x, y, z),
                        device_id_type=pl.DeviceIdType.MESH)

                    @pl.when(jnp.logical_not(is_self))
                    def _():
                        rn.wait_recv()
                        rs.wait_recv()

            for rn, rs in sends:
                rn.wait_send()
                rs.wait_send()

            mm = pstats[:, :, h0:h0 + HC, 0]
            ll = pstats[:, :, h0:h0 + HC, 1]
            m_t = jnp.maximum(mm[0], mm[1])
            c0 = jnp.exp(mm[0] - m_t)
            c1 = jnp.exp(mm[1] - m_t)
            l_t = ll[0] * c0 + ll[1] * c1
            nums = pnum[:, :, h0:h0 + HC].astype(jnp.float32)
            num_t = nums[0] * c0[..., None] + nums[1] * c1[..., None]
            out_r = num_t / l_t[..., None]
            for gg in range(G):
                for hh in range(HC):
                    out_ref[:, :, gg * HG + h0 + hh, :] = out_r[gg, hh]

        sends = [None] * (HG // HC)
        for h in range(HG):
            for c in loads[h]:
                c.wait()
            compute(h)
            if h == 0:
                pl.semaphore_wait(barrier, len(peers))
            if (h + 1) % HC == 0:
                sends[h // HC] = start_sends(h // HC)
        drain(0, sends[0])
        drain(1, sends[1])

    return pl.pallas_call(
        body,
        out_shape=jax.ShapeDtypeStruct((B, Sq, H, D), jnp.float32),
        in_specs=[
            pl.BlockSpec(memory_space=pltpu.HBM),
            pl.BlockSpec(memory_space=pltpu.HBM),
            pl.BlockSpec(memory_space=pltpu.HBM),
        ],
        out_specs=pl.BlockSpec(memory_space=pltpu.VMEM),
        scratch_shapes=[
            pltpu.VMEM((HG, B, Sq, D), jnp.float32),
            pltpu.VMEM((HG, B, Skv, D), jnp.float32),
            pltpu.VMEM((HG, B, Skv, D), jnp.float32),
            pltpu.SemaphoreType.DMA((3 * HG,)),
            pltpu.VMEM((2, G, HG, B, Sq, D), jnp.bfloat16),
            pltpu.VMEM((2, G, HG, 2, B, Sq), jnp.float32),
            pltpu.SemaphoreType.DMA((HG, 7)),
            pltpu.SemaphoreType.DMA((HG, 7)),
            pltpu.SemaphoreType.DMA((2, G, HG)),
            pltpu.SemaphoreType.DMA((2, G, HG)),
        ],
        compiler_params=pltpu.CompilerParams(
            collective_id=0, vmem_limit_bytes=100 * 1024 * 1024),
    )(Q, K, V)


# device time: 23135 ns/iter; 1.0468x vs baseline; 1.0468x over previous
import jax
import jax.numpy as jnp
from jax import lax
from jax.experimental import pallas as pl
from jax.experimental.pallas import tpu as pltpu


def kernel(Q, K, V):
    B, Sq, H, D = Q.shape
    _, Skv, _, _ = K.shape
    G = 4
    HG = H // G
    scale = D ** -0.5

    def body(q_hbm, k_hbm, v_hbm, out_ref,
             q_vmem, k_vmem, v_vmem, load_sems,
             pnum, pstats, nsend_sems, ssend_sems, nrecv_sems, srecv_sems):
        x = lax.axis_index("x")
        y = lax.axis_index("y")
        z = lax.axis_index("z")
        g = 2 * x + z

        barrier = pltpu.get_barrier_semaphore()
        peers = ((1 - x, y, z), (x, 1 - y, z), (x, y, 1 - z),
                 (1 - x, 1 - y, z), (1 - x, y, 1 - z), (x, 1 - y, 1 - z),
                 (1 - x, 1 - y, 1 - z))
        for nbr in peers:
            pl.semaphore_signal(barrier, inc=1, device_id=nbr,
                                device_id_type=pl.DeviceIdType.MESH)

        loads = []
        for h in range(HG):
            loads.append((
                pltpu.make_async_copy(
                    k_hbm.at[:, :, g * HG + h, :], k_vmem.at[h],
                    load_sems.at[h]),
                pltpu.make_async_copy(
                    v_hbm.at[:, :, g * HG + h, :], v_vmem.at[h],
                    load_sems.at[HG + h]),
                pltpu.make_async_copy(
                    q_hbm.at[:, :, g * HG + h, :], q_vmem.at[h],
                    load_sems.at[2 * HG + h]),
            ))
        for ld in loads:
            for c in ld:
                c.start()

        def compute(h):
            qb = q_vmem[h].astype(jnp.bfloat16)
            kb = k_vmem[h].astype(jnp.bfloat16)
            vb = v_vmem[h].astype(jnp.bfloat16)
            s = lax.dot_general(
                qb, kb, (((2,), (2,)), ((0,), (0,))),
                preferred_element_type=jnp.float32) * scale
            m = jnp.max(s, axis=-1)
            p = jnp.exp(s - m[..., None])
            l = jnp.sum(p, axis=-1)
            num = lax.dot_general(
                p.astype(jnp.bfloat16), vb, (((2,), (1,)), ((0,), (0,))),
                preferred_element_type=jnp.float32)
            pnum[y, g, h] = num.astype(jnp.bfloat16)
            pstats[y, g, h, 0] = m
            pstats[y, g, h, 1] = l

        HC = 1

        def start_sends(r):
            h0 = r * HC
            rdmas = []
            for j, dst in enumerate(peers):
                rn = pltpu.make_async_remote_copy(
                    pnum.at[y, g, pl.ds(h0, HC)],
                    pnum.at[y, g, pl.ds(h0, HC)],
                    nsend_sems.at[r, j], nrecv_sems.at[y, g, r],
                    device_id=dst, device_id_type=pl.DeviceIdType.MESH)
                rs = pltpu.make_async_remote_copy(
                    pstats.at[y, g, pl.ds(h0, HC)],
                    pstats.at[y, g, pl.ds(h0, HC)],
                    ssend_sems.at[r, j], srecv_sems.at[y, g, r],
                    device_id=dst, device_id_type=pl.DeviceIdType.MESH)
                rn.start()
                rs.start()
                rdmas.append((rn, rs))
            return rdmas

        def drain(r, sends):
            h0 = r * HC
            for yy in range(2):
                for gg in range(G):
                    is_self = jnp.logical_and(yy == y, gg == g)
                    rn = pltpu.make_async_remote_copy(
                        pnum.at[yy, gg, pl.ds(h0, HC)],
                        pnum.at[yy, gg, pl.ds(h0, HC)],
                        nsend_sems.at[r, 0], nrecv_sems.at[yy, gg, r],
                        device_id=(x, y, z),
                        device_id_type=pl.DeviceIdType.MESH)
                    rs = pltpu.make_async_remote_copy(
                        pstats.at[yy, gg, pl.ds(h0, HC)],
                        pstats.at[yy, gg, pl.ds(h0, HC)],
                        ssend_sems.at[r, 0], srecv_sems.at[yy, gg, r],
                        device_id=(x, y, z),
                        device_id_type=pl.DeviceIdType.MESH)

                    @pl.when(jnp.logical_not(is_self))
                    def _():
                        rn.wait_recv()
                        rs.wait_recv()

            for rn, rs in sends:
                rn.wait_send()
                rs.wait_send()

            mm = pstats[:, :, h0:h0 + HC, 0]
            ll = pstats[:, :, h0:h0 + HC, 1]
            m_t = jnp.maximum(mm[0], mm[1])
            c0 = jnp.exp(mm[0] - m_t)
            c1 = jnp.exp(mm[1] - m_t)
            l_t = ll[0] * c0 + ll[1] * c1
            nums = pnum[:, :, h0:h0 + HC].astype(jnp.float32)
            num_t = nums[0] * c0[..., None] + nums[1] * c1[..., None]
            out_r = num_t / l_t[..., None]
            for gg in range(G):
                for hh in range(HC):
                    out_ref[:, :, gg * HG + h0 + hh, :] = out_r[gg, hh]

        R = HG // HC
        sends = [None] * R
        for h in range(HG):
            for c in loads[h]:
                c.wait()
            compute(h)
            if h == 0:
                pl.semaphore_wait(barrier, len(peers))
            if (h + 1) % HC == 0:
                r = h // HC
                sends[r] = start_sends(r)
                if r >= 2:
                    drain(r - 2, sends[r - 2])
        for r in range(max(R - 2, 0), R):
            drain(r, sends[r])

    return pl.pallas_call(
        body,
        out_shape=jax.ShapeDtypeStruct((B, Sq, H, D), jnp.float32),
        in_specs=[
            pl.BlockSpec(memory_space=pltpu.HBM),
            pl.BlockSpec(memory_space=pltpu.HBM),
            pl.BlockSpec(memory_space=pltpu.HBM),
        ],
        out_specs=pl.BlockSpec(memory_space=pltpu.VMEM),
        scratch_shapes=[
            pltpu.VMEM((HG, B, Sq, D), jnp.float32),
            pltpu.VMEM((HG, B, Skv, D), jnp.float32),
            pltpu.VMEM((HG, B, Skv, D), jnp.float32),
            pltpu.SemaphoreType.DMA((3 * HG,)),
            pltpu.VMEM((2, G, HG, B, Sq, D), jnp.bfloat16),
            pltpu.VMEM((2, G, HG, 2, B, Sq), jnp.float32),
            pltpu.SemaphoreType.DMA((HG, 7)),
            pltpu.SemaphoreType.DMA((HG, 7)),
            pltpu.SemaphoreType.DMA((2, G, HG)),
            pltpu.SemaphoreType.DMA((2, G, HG)),
        ],
        compiler_params=pltpu.CompilerParams(
            collective_id=0, vmem_limit_bytes=100 * 1024 * 1024),
    )(Q, K, V)
